# baseline (speedup 1.0000x reference)
import dataclasses
import functools

import jax
import jax.numpy as jnp
from jax import lax
from jax.experimental import pallas as pl
from jax.experimental.pallas import tpu as pltpu
from jax.experimental.pallas import tpu_sc as plsc

RATIO = 0.1

_B1 = 11
_B2 = 11
_TC_BITS = 32 - _B1 - _B2


def _sc_select(xp, k):
    n, size = xp.shape
    nw = 32
    ppw = n // nw
    ppw_pad = max(8, ((ppw + 7) // 8) * 8)
    nb1 = 1 << _B1
    nb2 = 1 << _B2
    mesh = plsc.VectorSubcoreMesh(core_axis_name="c", subcore_axis_name="s")
    cp = pltpu.CompilerParams()
    if "needs_layout_passes" in pltpu.CompilerParams.__dataclass_fields__:
        cp = dataclasses.replace(cp, needs_layout_passes=False)

    @functools.partial(
        pl.kernel,
        out_type=jax.ShapeDtypeStruct((nw, ppw_pad), jnp.int32),
        mesh=mesh,
        compiler_params=cp,
        scratch_types=[
            pltpu.VMEM((size,), jnp.float32),
            pltpu.VMEM((max(nb1, nb2),), jnp.int32),
            pltpu.VMEM((ppw_pad,), jnp.int32),
        ],
    )
    def sel(x_hbm, lo_hbm, buf, hist, orow):
        w = lax.axis_index("s") * 2 + lax.axis_index("c")
        lanes = jnp.arange(16, dtype=jnp.int32)
        ones = jnp.ones((16,), jnp.int32)
        zeros16 = jnp.zeros((16,), jnp.int32)
        big = jnp.int32(0x7FFFFFFF)

        def keys_at(i):
            v = buf[pl.ds(i, 16)]
            b = plsc.bitcast(v, jnp.int32)
            return b ^ ((b >> 31) & jnp.int32(0x7FFFFFFF))

        def zero_hist(nbins):
            @pl.loop(0, nbins, step=64)
            def _(i):
                for u in range(4):
                    hist[pl.ds(i + u * 16, 16)] = zeros16

        def scan_hist(nbins, th, m_tot):
            def body(jj, car):
                r, cnt, mn = car
                h = hist[pl.ds(jj * 16, 16)]
                pc = plsc.cumsum(h)
                pexcl = (r + pc) - h
                ind = pexcl <= th
                cnt = cnt + jnp.where(ind, jnp.int32(1), jnp.int32(0))
                mn = jnp.minimum(mn, jnp.where(ind, big, pexcl))
                r = r + jnp.max(pc)
                return r, cnt, mn

            init = (jnp.int32(0), zeros16, jnp.full((16,), big, jnp.int32))
            _, cnt, mn = lax.fori_loop(0, nbins // 16, body, init)
            bsel = jnp.sum(cnt) - 1
            p_next = jnp.minimum(jnp.min(mn), m_tot)
            return bsel, p_next

        @pl.loop(0, ppw)
        def _plane(j):
            plane = w * ppw + j
            pltpu.sync_copy(x_hbm.at[plane], buf)

            zero_hist(nb1)

            @pl.loop(0, size, step=64)
            def _(i):
                for u in range(4):
                    key = keys_at(i + u * 16)
                    bkt = (key >> (32 - _B1)) + jnp.int32(nb1 // 2)
                    plsc.addupdate_scatter(hist, [bkt], ones)

            th1 = jnp.int32(size - k)
            b1, p_next1 = scan_hist(nb1, th1, jnp.int32(size))
            g_above = jnp.int32(size) - p_next1
            m1v = plsc.load_gather(hist, [jnp.broadcast_to(b1, (16,))])
            m1 = jnp.max(m1v)
            hi1 = b1 - jnp.int32(nb1 // 2)

            zero_hist(nb2)

            @pl.loop(0, size, step=64)
            def _(i):
                for u in range(4):
                    key = keys_at(i + u * 16)
                    memb = (key >> (32 - _B1)) == hi1
                    bkt = (key >> _TC_BITS) & jnp.int32(nb2 - 1)
                    plsc.addupdate_scatter(hist, [bkt], ones, mask=memb)

            th2 = (m1 + g_above) - jnp.int32(k)
            b2, _ = scan_hist(nb2, th2, m1)

            lo0 = (hi1 << (32 - _B1)) | (b2 << _TC_BITS)
            plsc.store_scatter(
                orow,
                [jnp.broadcast_to(j, (16,)).astype(jnp.int32)],
                jnp.broadcast_to(lo0, (16,)),
                mask=lanes == 0,
            )

        pltpu.sync_copy(orow, lo_hbm.at[w])

    return sel(xp)[:, :ppw].reshape(n)


def _tc_body(k, x_ref, lo_ref, o_ref):
    xb = x_ref[...]
    u = pltpu.bitcast(xb, jnp.uint32)
    neg = (u >> 31).astype(jnp.uint32)
    key = pltpu.bitcast(u ^ (neg * jnp.uint32(0x7FFFFFFF)), jnp.int32)

    p = xb.shape[0]
    kvec = jnp.full((p,), k, dtype=jnp.int32)
    lo = lo_ref[0, 0, :]

    def count_ge(t):
        m = key >= t[:, None, None]
        return jnp.sum(m.astype(jnp.int32), axis=(1, 2))

    for b in range(_TC_BITS - 1, -1, -1):
        t = lo | jnp.int32(1 << b)
        c = count_ge(t)
        lo = jnp.where(c >= kvec, t, lo)

    mask = key >= lo[:, None, None]
    o_ref[...] = jnp.where(mask, xb, jnp.float32(0.0))


def _tc_refine_mask(xp, lo, k, p_group):
    n, s, l = xp.shape
    grid = n // p_group
    lo3 = lo.reshape(grid, 1, p_group)
    return pl.pallas_call(
        functools.partial(_tc_body, k),
        grid=(grid,),
        in_specs=[
            pl.BlockSpec((p_group, s, l), lambda i: (i, 0, 0)),
            pl.BlockSpec((1, 1, p_group), lambda i: (i, 0, 0)),
        ],
        out_specs=pl.BlockSpec((p_group, s, l), lambda i: (i, 0, 0)),
        out_shape=jax.ShapeDtypeStruct((n, s, l), jnp.float32),
        compiler_params=pltpu.CompilerParams(
            dimension_semantics=("arbitrary",),
        ),
    )(xp, lo3)


def kernel(x):
    b, c, h, w = x.shape
    size = h * w
    k = int(RATIO * size)
    n = b * c
    lanes = 128
    assert size % lanes == 0 and size % 64 == 0 and n % 32 == 0
    s = size // lanes

    xflat = x.reshape(n, size)
    lo = _sc_select(xflat, k)

    xp = x.reshape(n, s, lanes)
    p_group = 32
    while n % p_group:
        p_group //= 2
    out = _tc_refine_mask(xp, lo, k, p_group)
    return out.reshape(b, c, h, w)
